# Initial kernel scaffold; baseline (speedup 1.0000x reference)
import functools

import jax
import jax.numpy as jnp
import numpy as np
from jax import lax
from jax.experimental import pallas as pl
from jax.experimental.pallas import tpu as pltpu
from jax.experimental.pallas import tpu_sc as plsc

B = 4
HW = 512 * 512
L = 1024
N = B * L
D = 16
IBLK = 128


@functools.lru_cache(maxsize=None)
def _gather_indices():
    key = jax.random.key(42)
    k_in, k_tg = jax.random.split(key)
    rows, lanes = [], []
    t = np.arange(N, dtype=np.int64)
    for k in (k_in, k_tg):
        r = np.asarray(jax.random.randint(k, (N,), 0, 256), dtype=np.int64)
        f = 256 * t + r
        b = f // (256 * L)
        c = (f % (256 * L)) // L
        p = f % L
        h = (p // 32) * 16 + c // 16
        w = (p % 32) * 16 + c % 16
        addr = b * HW + h * 512 + w
        rows.append(addr // D)
        lanes.append(addr % D)
    return (np.asarray(rows, dtype=np.int32),
            np.asarray(lanes, dtype=np.int32))


def _sc_gather(table_in, table_tg, rows, lanes):
    info = plsc.get_sparse_core_info()
    nw = info.num_cores * info.num_subcores
    bpw = N // nw
    nchunks = bpw // 16

    mesh = plsc.VectorSubcoreMesh(core_axis_name="c", subcore_axis_name="s")

    @functools.partial(
        pl.kernel,
        mesh=mesh,
        out_type=jax.ShapeDtypeStruct((2, N), jnp.float32),
        scratch_types=[
            pltpu.VMEM((bpw,), jnp.int32),
            pltpu.VMEM((bpw,), jnp.int32),
            pltpu.VMEM((bpw, D), jnp.float32),
            pltpu.VMEM((bpw,), jnp.float32),
            pltpu.SemaphoreType.DMA,
        ],
    )
    def k(tab_in, tab_tg, rows_hbm, lanes_hbm, out, rowv, lanev, rbuf, sbuf,
          sem):
        wid = lax.axis_index("s") * info.num_cores + lax.axis_index("c")
        base = wid * bpw
        for img, tab in ((0, tab_in), (1, tab_tg)):
            pltpu.sync_copy(rows_hbm.at[img, pl.ds(base, bpw)], rowv)
            pltpu.sync_copy(lanes_hbm.at[img, pl.ds(base, bpw)], lanev)
            pltpu.async_copy(tab.at[rowv], rbuf, sem).wait()
            for j in range(nchunks):
                rid = lax.iota(jnp.int32, 16) + 16 * j
                lid = lanev[pl.ds(16 * j, 16)]
                sbuf[pl.ds(16 * j, 16)] = plsc.load_gather(rbuf, [rid, lid])
            pltpu.sync_copy(sbuf, out.at[img, pl.ds(base, bpw)])

    return k(table_in, table_tg, rows, lanes)


def _pair_loss_body(xi_ref, ti_ref, xj_ref, tj_ref, out_ref, acc_ref):
    bi = pl.program_id(0)
    ki = pl.program_id(1)
    step = bi * (L // IBLK) + ki

    @pl.when(step == 0)
    def _init():
        acc_ref[0] = 0.0
        acc_ref[1] = 0.0
        acc_ref[2] = 0.0
        acc_ref[3] = 0.0

    xi = xi_ref[0, :].reshape(IBLK, 1)
    ti = ti_ref[0, :].reshape(IBLK, 1)
    xj = xj_ref[0, :].reshape(1, L)
    tj = tj_ref[0, :].reshape(1, L)

    row = lax.broadcasted_iota(jnp.int32, (IBLK, L), 0) + ki * IBLK
    col = lax.broadcasted_iota(jnp.int32, (IBLK, L), 1)
    triu = col > row

    dt = ti - tj
    dx = xi - xj
    is_eq = jnp.abs(dt) < 0.02
    eq = jnp.logical_and(is_eq, triu)
    ne = jnp.logical_and(jnp.logical_not(is_eq), triu)

    om = jnp.where(dt > 0, 1.0, -1.0)
    of = 1.0 + jnp.exp(-om * dx)
    wk = 1.0 - 1.0 / of
    ordv = (wk * wk) * jnp.log(of)

    zero = jnp.zeros_like(dx)
    acc_ref[0] += jnp.sum(jnp.where(ne, ordv, zero))
    acc_ref[1] += jnp.sum(jnp.where(ne, 1.0, 0.0))
    acc_ref[2] += jnp.sum(jnp.where(eq, dx * dx, zero))
    acc_ref[3] += jnp.sum(jnp.where(eq, 1.0, 0.0))

    @pl.when(step == B * (L // IBLK) - 1)
    def _fin():
        ord_mean = acc_ref[0] / jnp.maximum(acc_ref[1], 1.0)
        mse_mean = acc_ref[2] / jnp.maximum(acc_ref[3], 1.0)
        out_ref[0, 0] = ord_mean + mse_mean


def _pair_loss(xs, ts, interpret=False):
    grid = (B, L // IBLK)
    out = pl.pallas_call(
        _pair_loss_body,
        grid=grid,
        in_specs=[
            pl.BlockSpec((1, IBLK), lambda b, k: (b, k)),
            pl.BlockSpec((1, IBLK), lambda b, k: (b, k)),
            pl.BlockSpec((1, L), lambda b, k: (b, 0)),
            pl.BlockSpec((1, L), lambda b, k: (b, 0)),
        ],
        out_specs=pl.BlockSpec((1, 1), lambda b, k: (0, 0)),
        out_shape=jax.ShapeDtypeStruct((1, 1), jnp.float32),
        scratch_shapes=[pltpu.SMEM((4,), jnp.float32)],
        interpret=interpret,
    )(xs, ts, xs, ts)
    return out.reshape(())


def kernel(input, target):
    rows, lanes = _gather_indices()
    table_in = input.reshape(HW * B // D, D)
    table_tg = target.reshape(HW * B // D, D)
    samples = _sc_gather(table_in, table_tg,
                         jnp.asarray(rows), jnp.asarray(lanes))
    xs = samples[0].reshape(B, L)
    ts = samples[1].reshape(B, L)
    return _pair_loss(xs, ts)


# speedup vs baseline: 59.9887x; 59.9887x over previous
import functools

import jax
import jax.numpy as jnp
import numpy as np
from jax import lax
from jax.experimental import pallas as pl
from jax.experimental.pallas import tpu as pltpu
from jax.experimental.pallas import tpu_sc as plsc

B = 4
HW = 512 * 512
L = 1024
N = B * L
D = 128
IBLK = 128


def _threefry2x32_raw(ks, c1, c2):
    rotations = ((13, 15, 26, 6), (17, 29, 16, 24))

    def rotl(x, d):
        return ((x << np.uint32(d)) | (x >> np.uint32(32 - d))).astype(
            np.uint32)

    with np.errstate(over="ignore"):
        k0, k1 = np.uint32(ks[0]), np.uint32(ks[1])
        k2 = np.uint32(k0 ^ k1 ^ np.uint32(0x1BD11BDA))
        keys = (k0, k1, k2)
        x = [c1.astype(np.uint32) + k0, c2.astype(np.uint32) + k1]
        for i in range(5):
            for rot in rotations[i % 2]:
                x[0] = (x[0] + x[1]).astype(np.uint32)
                x[1] = x[0] ^ rotl(x[1], rot)
            x[0] = (x[0] + keys[(i + 1) % 3]).astype(np.uint32)
            x[1] = (x[1] + keys[(i + 2) % 3] + np.uint32(i + 1)).astype(
                np.uint32)
    return x[0], x[1]


def _np_split(key):
    b1, b2 = _threefry2x32_raw(key, np.zeros(2, np.uint32),
                               np.arange(2, dtype=np.uint32))
    return np.stack([b1, b2], axis=1)


def _np_random_bits(key, n):
    b1, b2 = _threefry2x32_raw(key, np.zeros(n, np.uint32),
                               np.arange(n, dtype=np.uint32))
    return b1 ^ b2


def _np_randint_mod(key, n, span):
    sub = _np_split(key)
    y = _np_random_bits(sub[0], n)
    z = _np_random_bits(sub[1], n)
    span = np.uint32(span)
    mult = ((np.uint64(65536 % span) ** 2) % np.uint64(span)).astype(np.uint32)
    with np.errstate(over="ignore"):
        r = ((y % span) * mult + (z % span)) % span
    return r.astype(np.int64)


@functools.lru_cache(maxsize=None)
def _gather_indices():
    root = np.array([0, 42], dtype=np.uint32)
    k_in, k_tg = _np_split(root)
    rows, lanes = [], []
    t = np.arange(N, dtype=np.int64)
    for k in (k_in, k_tg):
        r = _np_randint_mod(k, N, 256)
        f = 256 * t + r
        b = f // (256 * L)
        c = (f % (256 * L)) // L
        p = f % L
        h = (p // 32) * 16 + c // 16
        w = (p % 32) * 16 + c % 16
        addr = b * HW + h * 512 + w
        rows.append(addr // D)
        lanes.append(addr % D)
    return (np.asarray(rows, dtype=np.int32),
            np.asarray(lanes, dtype=np.int32))


def _sc_gather(table_in, table_tg, rows):
    info = plsc.get_sparse_core_info()
    nw = info.num_cores * info.num_subcores
    bpw = N // nw

    mesh = plsc.VectorSubcoreMesh(core_axis_name="c", subcore_axis_name="s")

    @functools.partial(
        pl.kernel,
        mesh=mesh,
        out_type=jax.ShapeDtypeStruct((2, N, D), jnp.float32),
        scratch_types=[
            pltpu.VMEM((bpw,), jnp.int32),
            pltpu.VMEM((bpw, D), jnp.float32),
            pltpu.SemaphoreType.DMA,
        ],
    )
    def k(tab_in, tab_tg, rows_hbm, out, rowv, rbuf, sem):
        wid = lax.axis_index("s") * info.num_cores + lax.axis_index("c")
        base = wid * bpw
        for img, tab in ((0, tab_in), (1, tab_tg)):
            pltpu.sync_copy(rows_hbm.at[img, pl.ds(base, bpw)], rowv)
            pltpu.async_copy(tab.at[rowv], rbuf, sem).wait()
            pltpu.sync_copy(rbuf, out.at[img, pl.ds(base, bpw)])

    return k(table_in, table_tg, rows)


def _lane_select_body(rows_ref, lane_ref, out_ref):
    n = rows_ref.shape[0]
    oh = (lax.broadcasted_iota(jnp.int32, (n, D), 1)
          == lane_ref[:].reshape(n, 1))
    out_ref[:] = jnp.sum(jnp.where(oh, rows_ref[:, :], 0.0), axis=1)


def _lane_select(rows, lanes, interpret=False):
    nt = rows.shape[0]
    blk = 512
    return pl.pallas_call(
        _lane_select_body,
        grid=(nt // blk,),
        in_specs=[
            pl.BlockSpec((blk, D), lambda i: (i, 0)),
            pl.BlockSpec((blk,), lambda i: (i,)),
        ],
        out_specs=pl.BlockSpec((blk,), lambda i: (i,)),
        out_shape=jax.ShapeDtypeStruct((nt,), jnp.float32),
        interpret=interpret,
    )(rows, lanes)


def _pair_loss_body(xi_ref, ti_ref, xj_ref, tj_ref, out_ref, acc_ref):
    bi = pl.program_id(0)
    ki = pl.program_id(1)
    step = bi * (L // IBLK) + ki

    @pl.when(step == 0)
    def _init():
        acc_ref[0] = 0.0
        acc_ref[1] = 0.0
        acc_ref[2] = 0.0
        acc_ref[3] = 0.0

    xi = xi_ref[0, 0, :].reshape(IBLK, 1)
    ti = ti_ref[0, 0, :].reshape(IBLK, 1)
    xj = xj_ref[0, 0, :].reshape(1, L)
    tj = tj_ref[0, 0, :].reshape(1, L)

    row = lax.broadcasted_iota(jnp.int32, (IBLK, L), 0) + ki * IBLK
    col = lax.broadcasted_iota(jnp.int32, (IBLK, L), 1)
    triu = col > row

    dt = ti - tj
    dx = xi - xj
    is_eq = jnp.abs(dt) < 0.02
    eq = jnp.logical_and(is_eq, triu)
    ne = jnp.logical_and(jnp.logical_not(is_eq), triu)

    om = jnp.where(dt > 0, 1.0, -1.0)
    of = 1.0 + jnp.exp(-om * dx)
    wk = 1.0 - 1.0 / of
    ordv = (wk * wk) * jnp.log(of)

    zero = jnp.zeros_like(dx)
    acc_ref[0] += jnp.sum(jnp.where(ne, ordv, zero))
    acc_ref[1] += jnp.sum(jnp.where(ne, 1.0, 0.0))
    acc_ref[2] += jnp.sum(jnp.where(eq, dx * dx, zero))
    acc_ref[3] += jnp.sum(jnp.where(eq, 1.0, 0.0))

    @pl.when(step == B * (L // IBLK) - 1)
    def _fin():
        ord_mean = acc_ref[0] / jnp.maximum(acc_ref[1], 1.0)
        mse_mean = acc_ref[2] / jnp.maximum(acc_ref[3], 1.0)
        out_ref[:, :] = jnp.reshape(ord_mean + mse_mean, (1, 1))


def _pair_loss(xs, ts, interpret=False):
    grid = (B, L // IBLK)
    nchunk = L // IBLK
    ispec = pl.BlockSpec((1, 1, IBLK), lambda b, k: (b * nchunk + k, 0, 0))
    jspec = pl.BlockSpec((1, 1, L), lambda b, k: (b, 0, 0))
    out = pl.pallas_call(
        _pair_loss_body,
        grid=grid,
        in_specs=[ispec, ispec, jspec, jspec],
        out_specs=pl.BlockSpec((1, 1), lambda b, k: (0, 0)),
        out_shape=jax.ShapeDtypeStruct((1, 1), jnp.float32),
        scratch_shapes=[pltpu.SMEM((4,), jnp.float32)],
        interpret=interpret,
    )(xs.reshape(B * nchunk, 1, IBLK), ts.reshape(B * nchunk, 1, IBLK),
      xs.reshape(B, 1, L), ts.reshape(B, 1, L))
    return out.reshape(())


def kernel(input, target):
    rows, lanes = _gather_indices()
    table_in = input.reshape(HW * B // D, D)
    table_tg = target.reshape(HW * B // D, D)
    gathered = _sc_gather(table_in, table_tg, jnp.asarray(rows))
    samples = _lane_select(gathered.reshape(2 * N, D),
                           jnp.asarray(lanes).reshape(2 * N))
    return _pair_loss(samples[:N].reshape(B, L), samples[N:].reshape(B, L))


# speedup vs baseline: 61.9411x; 1.0325x over previous
import functools

import jax
import jax.numpy as jnp
import numpy as np
from jax import lax
from jax.experimental import pallas as pl
from jax.experimental.pallas import tpu as pltpu
from jax.experimental.pallas import tpu_sc as plsc

B = 4
HW = 512 * 512
L = 1024
N = B * L
D = 128
IBLK = 256


def _threefry2x32_raw(ks, c1, c2):
    rotations = ((13, 15, 26, 6), (17, 29, 16, 24))

    def rotl(x, d):
        return ((x << np.uint32(d)) | (x >> np.uint32(32 - d))).astype(
            np.uint32)

    with np.errstate(over="ignore"):
        k0, k1 = np.uint32(ks[0]), np.uint32(ks[1])
        k2 = np.uint32(k0 ^ k1 ^ np.uint32(0x1BD11BDA))
        keys = (k0, k1, k2)
        x = [c1.astype(np.uint32) + k0, c2.astype(np.uint32) + k1]
        for i in range(5):
            for rot in rotations[i % 2]:
                x[0] = (x[0] + x[1]).astype(np.uint32)
                x[1] = x[0] ^ rotl(x[1], rot)
            x[0] = (x[0] + keys[(i + 1) % 3]).astype(np.uint32)
            x[1] = (x[1] + keys[(i + 2) % 3] + np.uint32(i + 1)).astype(
                np.uint32)
    return x[0], x[1]


def _np_split(key):
    b1, b2 = _threefry2x32_raw(key, np.zeros(2, np.uint32),
                               np.arange(2, dtype=np.uint32))
    return np.stack([b1, b2], axis=1)


def _np_random_bits(key, n):
    b1, b2 = _threefry2x32_raw(key, np.zeros(n, np.uint32),
                               np.arange(n, dtype=np.uint32))
    return b1 ^ b2


def _np_randint_mod(key, n, span):
    sub = _np_split(key)
    y = _np_random_bits(sub[0], n)
    z = _np_random_bits(sub[1], n)
    span = np.uint32(span)
    mult = ((np.uint64(65536 % span) ** 2) % np.uint64(span)).astype(np.uint32)
    with np.errstate(over="ignore"):
        r = ((y % span) * mult + (z % span)) % span
    return r.astype(np.int64)


@functools.lru_cache(maxsize=None)
def _gather_indices():
    root = np.array([0, 42], dtype=np.uint32)
    k_in, k_tg = _np_split(root)
    rows, lanes = [], []
    t = np.arange(N, dtype=np.int64)
    for k in (k_in, k_tg):
        r = _np_randint_mod(k, N, 256)
        f = 256 * t + r
        b = f // (256 * L)
        c = (f % (256 * L)) // L
        p = f % L
        h = (p // 32) * 16 + c // 16
        w = (p % 32) * 16 + c % 16
        addr = b * HW + h * 512 + w
        rows.append(addr // D)
        lanes.append(addr % D)
    return (np.asarray(rows, dtype=np.int32),
            np.asarray(lanes, dtype=np.int32))


def _sc_gather(table_in, table_tg, rows):
    info = plsc.get_sparse_core_info()
    nw = info.num_cores * info.num_subcores
    bpw = N // nw

    mesh = plsc.VectorSubcoreMesh(core_axis_name="c", subcore_axis_name="s")

    @functools.partial(
        pl.kernel,
        mesh=mesh,
        out_type=jax.ShapeDtypeStruct((2, N, D), jnp.float32),
        scratch_types=[
            pltpu.VMEM((bpw,), jnp.int32),
            pltpu.VMEM((bpw, D), jnp.float32),
            pltpu.SemaphoreType.DMA,
        ],
    )
    def k(tab_in, tab_tg, rows_hbm, out, rowv, rbuf, sem):
        wid = lax.axis_index("s") * info.num_cores + lax.axis_index("c")
        base = wid * bpw
        for img, tab in ((0, tab_in), (1, tab_tg)):
            pltpu.sync_copy(rows_hbm.at[img, pl.ds(base, bpw)], rowv)
            pltpu.async_copy(tab.at[rowv], rbuf, sem).wait()
            pltpu.sync_copy(rbuf, out.at[img, pl.ds(base, bpw)])

    return k(table_in, table_tg, rows)


def _lane_select_body(rows_ref, lane_ref, out_ref):
    n = rows_ref.shape[0]
    oh = (lax.broadcasted_iota(jnp.int32, (n, D), 1)
          == lane_ref[:].reshape(n, 1))
    out_ref[:] = jnp.sum(jnp.where(oh, rows_ref[:, :], 0.0), axis=1)


def _lane_select(rows, lanes, interpret=False):
    nt = rows.shape[0]
    blk = 512
    return pl.pallas_call(
        _lane_select_body,
        grid=(nt // blk,),
        in_specs=[
            pl.BlockSpec((blk, D), lambda i: (i, 0)),
            pl.BlockSpec((blk,), lambda i: (i,)),
        ],
        out_specs=pl.BlockSpec((blk,), lambda i: (i,)),
        out_shape=jax.ShapeDtypeStruct((nt,), jnp.float32),
        interpret=interpret,
    )(rows, lanes)


NTILE = L // IBLK
NPAIR = NTILE * (NTILE + 1) // 2
_TI = [0, 1, 2, 3, 0, 0, 0, 1, 1, 2]
_TJ = [0, 1, 2, 3, 1, 2, 3, 2, 3, 3]
CNT_OFF = float(B * 6 * IBLK * IBLK)
CNT_DIAG = float(B * 4 * IBLK * IBLK)
NDIAG = float(B * L)


def _tile_idx(p):
    q = p - 4
    ti_off = (q >= 3).astype(jnp.int32) + (q >= 5).astype(jnp.int32)
    tj_off = jnp.where(q < 3, q + 1, jnp.where(q < 5, q - 1, 3))
    ti = jnp.where(p < 4, p, ti_off)
    tj = jnp.where(p < 4, p, tj_off)
    return ti, tj


def _pair_loss_body(xi_ref, ti_ref, xj_ref, tj_ref, out_ref, acc_ref):
    bi = pl.program_id(0)
    p = pl.program_id(1)
    step = bi * NPAIR + p

    @pl.when(step == 0)
    def _init():
        for i in range(6):
            acc_ref[i] = 0.0

    xi = xi_ref[0, 0, :].reshape(IBLK, 1)
    ti = ti_ref[0, 0, :].reshape(IBLK, 1)
    xj = xj_ref[0, 0, :].reshape(1, IBLK)
    tj = tj_ref[0, 0, :].reshape(1, IBLK)

    dt = ti - tj
    dx = xi - xj
    eqf = jnp.where(jnp.abs(dt) < 0.02, 1.0, 0.0)

    e = jnp.exp(jnp.where(dt > 0, -dx, dx))
    of = 1.0 + e
    wk = 1.0 - 1.0 / of
    ordv = (wk * wk) * jnp.log(of)

    s_ord = jnp.sum((1.0 - eqf) * ordv)
    s_mse = jnp.sum(eqf * (dx * dx))
    s_ef = jnp.sum(eqf)

    @pl.when(p < 4)
    def _diag():
        acc_ref[0] += s_ord
        acc_ref[1] += s_mse
        acc_ref[2] += s_ef

    @pl.when(p >= 4)
    def _off():
        acc_ref[3] += s_ord
        acc_ref[4] += s_mse
        acc_ref[5] += s_ef

    @pl.when(step == B * NPAIR - 1)
    def _fin():
        t_ord = acc_ref[3] + 0.5 * acc_ref[0]
        t_mse = acc_ref[4] + 0.5 * acc_ref[1]
        t_ef = acc_ref[5] + 0.5 * (acc_ref[2] - NDIAG)
        t_nf = (CNT_OFF - acc_ref[5]) + 0.5 * (CNT_DIAG - acc_ref[2])
        ord_mean = t_ord / jnp.maximum(t_nf, 1.0)
        mse_mean = t_mse / jnp.maximum(t_ef, 1.0)
        out_ref[:, :] = jnp.reshape(ord_mean + mse_mean, (1, 1))


def _pair_loss(xs, ts, interpret=False):
    grid = (B, NPAIR)

    def imap(b, p):
        ti, _ = _tile_idx(p)
        return b * NTILE + ti, 0, 0

    def jmap(b, p):
        _, tj = _tile_idx(p)
        return b * NTILE + tj, 0, 0

    ispec = pl.BlockSpec((1, 1, IBLK), imap)
    jspec = pl.BlockSpec((1, 1, IBLK), jmap)
    xc = xs.reshape(B * NTILE, 1, IBLK)
    tc = ts.reshape(B * NTILE, 1, IBLK)
    out = pl.pallas_call(
        _pair_loss_body,
        grid=grid,
        in_specs=[ispec, ispec, jspec, jspec],
        out_specs=pl.BlockSpec((1, 1), lambda b, p: (0, 0)),
        out_shape=jax.ShapeDtypeStruct((1, 1), jnp.float32),
        scratch_shapes=[pltpu.SMEM((6,), jnp.float32)],
        interpret=interpret,
    )(xc, tc, xc, tc)
    return out.reshape(())


def kernel(input, target):
    rows, lanes = _gather_indices()
    table_in = input.reshape(HW * B // D, D)
    table_tg = target.reshape(HW * B // D, D)
    gathered = _sc_gather(table_in, table_tg, jnp.asarray(rows))
    samples = _lane_select(gathered.reshape(2 * N, D),
                           jnp.asarray(lanes).reshape(2 * N))
    return _pair_loss(samples[:N].reshape(B, L), samples[N:].reshape(B, L))


# speedup vs baseline: 92.8954x; 1.4997x over previous
import functools

import jax
import jax.numpy as jnp
import numpy as np
from jax import lax
from jax.experimental import pallas as pl
from jax.experimental.pallas import tpu as pltpu
from jax.experimental.pallas import tpu_sc as plsc

B = 4
HW = 512 * 512
L = 1024
N = B * L
D = 128
IBLK = 256


def _threefry2x32_raw(ks, c1, c2):
    rotations = ((13, 15, 26, 6), (17, 29, 16, 24))

    def rotl(x, d):
        return ((x << np.uint32(d)) | (x >> np.uint32(32 - d))).astype(
            np.uint32)

    with np.errstate(over="ignore"):
        k0, k1 = np.uint32(ks[0]), np.uint32(ks[1])
        k2 = np.uint32(k0 ^ k1 ^ np.uint32(0x1BD11BDA))
        keys = (k0, k1, k2)
        x = [c1.astype(np.uint32) + k0, c2.astype(np.uint32) + k1]
        for i in range(5):
            for rot in rotations[i % 2]:
                x[0] = (x[0] + x[1]).astype(np.uint32)
                x[1] = x[0] ^ rotl(x[1], rot)
            x[0] = (x[0] + keys[(i + 1) % 3]).astype(np.uint32)
            x[1] = (x[1] + keys[(i + 2) % 3] + np.uint32(i + 1)).astype(
                np.uint32)
    return x[0], x[1]


def _np_split(key):
    b1, b2 = _threefry2x32_raw(key, np.zeros(2, np.uint32),
                               np.arange(2, dtype=np.uint32))
    return np.stack([b1, b2], axis=1)


def _np_random_bits(key, n):
    b1, b2 = _threefry2x32_raw(key, np.zeros(n, np.uint32),
                               np.arange(n, dtype=np.uint32))
    return b1 ^ b2


def _np_randint_mod(key, n, span):
    sub = _np_split(key)
    y = _np_random_bits(sub[0], n)
    z = _np_random_bits(sub[1], n)
    span = np.uint32(span)
    mult = ((np.uint64(65536 % span) ** 2) % np.uint64(span)).astype(np.uint32)
    with np.errstate(over="ignore"):
        r = ((y % span) * mult + (z % span)) % span
    return r.astype(np.int64)


@functools.lru_cache(maxsize=None)
def _gather_indices():
    root = np.array([0, 42], dtype=np.uint32)
    k_in, k_tg = _np_split(root)
    rows, lanes = [], []
    t = np.arange(N, dtype=np.int64)
    for k in (k_in, k_tg):
        r = _np_randint_mod(k, N, 256)
        f = 256 * t + r
        b = f // (256 * L)
        c = (f % (256 * L)) // L
        p = f % L
        h = (p // 32) * 16 + c // 16
        w = (p % 32) * 16 + c % 16
        addr = b * HW + h * 512 + w
        rows.append(addr // D)
        lanes.append(addr % D)
    return (np.asarray(rows, dtype=np.int32),
            np.asarray(lanes, dtype=np.int32))


def _sc_gather(table_in, table_tg, rows):
    info = plsc.get_sparse_core_info()
    nw = info.num_cores * info.num_subcores
    bpw = N // nw

    mesh = plsc.VectorSubcoreMesh(core_axis_name="c", subcore_axis_name="s")

    @functools.partial(
        pl.kernel,
        mesh=mesh,
        out_type=jax.ShapeDtypeStruct((2, N, D), jnp.float32),
        scratch_types=[
            pltpu.VMEM((bpw,), jnp.int32),
            pltpu.VMEM((bpw, D), jnp.float32),
            pltpu.SemaphoreType.DMA,
        ],
    )
    def k(tab_in, tab_tg, rows_hbm, out, rowv, rbuf, sem):
        wid = lax.axis_index("s") * info.num_cores + lax.axis_index("c")
        base = wid * bpw
        for img, tab in ((0, tab_in), (1, tab_tg)):
            pltpu.sync_copy(rows_hbm.at[img, pl.ds(base, bpw)], rowv)
            pltpu.async_copy(tab.at[rowv], rbuf, sem).wait()
            pltpu.sync_copy(rbuf, out.at[img, pl.ds(base, bpw)])

    return k(table_in, table_tg, rows)


NTILE = L // IBLK
_TILE_PAIRS = ([(t, t, True) for t in range(NTILE)]
               + [(i, j, False) for i in range(NTILE)
                  for j in range(i + 1, NTILE)])
CNT_OFF = float(B * (NTILE * (NTILE - 1) // 2) * IBLK * IBLK)
CNT_DIAG = float(B * NTILE * IBLK * IBLK)
NDIAG = float(B * L)


@functools.lru_cache(maxsize=None)
def _lane_onehot():
    _, lanes = _gather_indices()
    oh = np.zeros((2 * N, D), dtype=np.float32)
    oh[np.arange(2 * N), lanes.reshape(2 * N)] = 1.0
    return oh


def _fused_loss_body(rows_ref, oh_ref, out_ref):
    s = jnp.sum(rows_ref[:, :] * oh_ref[:, :], axis=1)
    xs = s[:N].reshape(B, L)
    ts = s[N:].reshape(B, L)

    acc = {True: [0.0, 0.0, 0.0], False: [0.0, 0.0, 0.0]}
    for ti, tj, is_diag in _TILE_PAIRS:
        xi = xs[:, ti * IBLK:(ti + 1) * IBLK][:, :, None]
        ui = ts[:, ti * IBLK:(ti + 1) * IBLK][:, :, None]
        xj = xs[:, tj * IBLK:(tj + 1) * IBLK][:, None, :]
        uj = ts[:, tj * IBLK:(tj + 1) * IBLK][:, None, :]

        dt = ui - uj
        dx = xi - xj
        is_eq = jnp.abs(dt) < 0.02

        e = jnp.exp(jnp.where(dt > 0, -dx, dx))
        of = 1.0 + e
        wk = 1.0 - 1.0 / of
        ordv = (wk * wk) * jnp.log(of)

        a = acc[is_diag]
        a[0] += jnp.sum(jnp.where(is_eq, 0.0, ordv))
        a[1] += jnp.sum(jnp.where(is_eq, dx * dx, 0.0))
        a[2] += jnp.sum(jnp.where(is_eq, 1.0, 0.0))

    dacc, oacc = acc[True], acc[False]
    t_ord = oacc[0] + 0.5 * dacc[0]
    t_mse = oacc[1] + 0.5 * dacc[1]
    t_ef = oacc[2] + 0.5 * (dacc[2] - NDIAG)
    t_nf = (CNT_OFF - oacc[2]) + 0.5 * (CNT_DIAG - dacc[2])
    ord_mean = t_ord / jnp.maximum(t_nf, 1.0)
    mse_mean = t_mse / jnp.maximum(t_ef, 1.0)
    out_ref[:, :] = jnp.reshape(ord_mean + mse_mean, (1, 1))


def _fused_loss(rows, interpret=False):
    out = pl.pallas_call(
        _fused_loss_body,
        out_shape=jax.ShapeDtypeStruct((1, 1), jnp.float32),
        interpret=interpret,
    )(rows, jnp.asarray(_lane_onehot()))
    return out.reshape(())


def kernel(input, target):
    rows, lanes = _gather_indices()
    table_in = input.reshape(HW * B // D, D)
    table_tg = target.reshape(HW * B // D, D)
    gathered = _sc_gather(table_in, table_tg, jnp.asarray(rows))
    return _fused_loss(gathered.reshape(2 * N, D))
